# Initial kernel scaffold; baseline (speedup 1.0000x reference)
import jax
import jax.numpy as jnp
from jax.experimental import pallas as pl


def kernel(h_dag, obs_ptr, W, b):
    raise NotImplementedError("write your pallas kernel here")


# speedup vs baseline: 38.6695x; 38.6695x over previous
import functools

import jax
import jax.numpy as jnp
from jax import lax
from jax.experimental import pallas as pl
from jax.experimental.pallas import tpu as pltpu
from jax.experimental.pallas import tpu_sc as plsc

N = 131072
B = 16384
D = 1024

NC = 2
NS = 16
NW = NC * NS
L = 16
CG = D // L
HB = B // 2
SBW = HB // NW
CR = 48
OS = 16
PTR_CHUNK = 344
PTR_PAD = 16472
CH = CG // 2


def _sread(ref, i):
  return ref[pl.ds(i, L)][0]


def _segmax_body(h_hbm, ptr_hbm, out_hbm, ptr_v, buf, oseg, sem, seg_lo=0):
  cid = lax.axis_index("c")
  sid = lax.axis_index("s")
  wid = sid * NC + cid
  s0 = wid * SBW

  pltpu.sync_copy(ptr_hbm.at[pl.ds(seg_lo + s0, PTR_CHUNK)], ptr_v)

  r0 = _sread(ptr_v, 0)
  r1 = _sread(ptr_v, SBW)

  neg = jnp.full((L,), -jnp.inf, dtype=jnp.float32)

  a0 = pl.multiple_of((r0 // 8) * 8, 8)
  nchunks = (r1 - a0 + CR - 1) // CR

  def chunk_base(c):
    return pl.multiple_of(jnp.minimum(a0 + c * CR, N - CR), 8)

  def chunk_dst(c):
    return buf.at[pl.ds(jnp.bitwise_and(c, 1) * CR, CR)]

  def issue(c):
    pltpu.async_copy(h_hbm.at[pl.ds(chunk_base(c), CR)], chunk_dst(c), sem)

  issue(jnp.int32(0))

  def chunk_body(c, s):
    pltpu.make_async_copy(
        h_hbm.at[pl.ds(chunk_base(c), CR)], chunk_dst(c), sem).wait()

    @pl.when(c + 1 < nchunks)
    def _():
      issue(c + 1)

    gridpos = a0 + c * CR
    rstart = jnp.maximum(r0, gridpos)
    rend = jnp.minimum(r1, gridpos + CR)
    boff = jnp.bitwise_and(c, 1) * CR + (rstart - chunk_base(c))

    lo = s
    vlo = _sread(ptr_v, s)
    for st in (32, 16, 8, 4, 2, 1):
      probe = lo + st
      pv = _sread(ptr_v, probe)
      ok = pv <= rend
      lo = jnp.where(ok, probe, lo)
      vlo = jnp.where(ok, pv, vlo)
    nb = lo - s
    nruns = nb + jnp.where(vlo < rend, jnp.int32(1), jnp.int32(0))

    def do_run(sk, q0, q1, pend, cont):
      t = jnp.bitwise_and(sk, OS - 1)
      row0 = boff + (q0 - rstart)
      nrows = q1 - q0

      for h in range(2):
        hof = h * CH * L
        if cont is None:
          acc = (neg,) * CH
        else:
          acc = tuple(
              jnp.where(cont, oseg[t, pl.ds(hof + cg * L, L)], neg)
              for cg in range(CH))

        def rb(i, a):
          return tuple(
              jnp.maximum(a[cg], buf[row0 + i, pl.ds(hof + cg * L, L)])
              for cg in range(CH))

        acc = lax.fori_loop(0, nrows, rb, acc)
        for cg in range(CH):
          oseg[t, pl.ds(hof + cg * L, L)] = acc[cg]

      done_group = jnp.logical_and(
          q1 == pend, jnp.bitwise_and(sk + 1, OS - 1) == 0)

      @pl.when(done_group)
      def _():
        gstart = pl.multiple_of(s0 + sk + 1 - OS, OS)
        pltpu.sync_copy(oseg, out_hbm.at[pl.ds(gstart, OS)])

    pstart0 = _sread(ptr_v, s)
    pend0 = _sread(ptr_v, s + 1)
    do_run(s, rstart, jnp.minimum(rend, pend0), pend0, pstart0 < rstart)

    def run_body(k, _):
      sk = s + k
      q0 = _sread(ptr_v, sk)
      pend = _sread(ptr_v, sk + 1)
      do_run(sk, q0, jnp.minimum(rend, pend), pend, None)
      return 0

    lax.fori_loop(1, nruns, run_body, 0)
    return s + nb

  lax.fori_loop(0, nchunks, chunk_body, jnp.int32(0))


def _segmax_sc(h_dag, ptr_pad, seg_lo):
  mesh = plsc.VectorSubcoreMesh(core_axis_name="c", subcore_axis_name="s")
  f = pl.kernel(
      functools.partial(_segmax_body, seg_lo=seg_lo),
      out_type=jax.ShapeDtypeStruct((HB, D), jnp.float32),
      mesh=mesh,
      scratch_types=[
          pltpu.VMEM((PTR_CHUNK,), jnp.int32),
          pltpu.VMEM((2 * CR, D), jnp.float32),
          pltpu.VMEM((OS, D), jnp.float32),
          pltpu.SemaphoreType.DMA,
      ],
      name=f"segmax_sc_{seg_lo}",
  )
  return f(h_dag, ptr_pad)


TM = 1024


def _mm_body(x_ref, w_ref, b_ref, o_ref):
  acc = lax.dot_general(
      x_ref[...], w_ref[...],
      (((1,), (1,)), ((), ())),
      preferred_element_type=jnp.float32,
  )
  o_ref[...] = acc + b_ref[...]


def _linear_tc(x, w, b2d):
  return pl.pallas_call(
      _mm_body,
      grid=(HB // TM,),
      in_specs=[
          pl.BlockSpec((TM, D), lambda i: (i, 0)),
          pl.BlockSpec((D, D), lambda i: (0, 0)),
          pl.BlockSpec((1, D), lambda i: (0, 0)),
      ],
      out_specs=pl.BlockSpec((TM, D), lambda i: (i, 0)),
      out_shape=jax.ShapeDtypeStruct((HB, D), jnp.float32),
  )(x, w, b2d)


def _mm2_body(y1_ref, x2_ref, w_ref, b_ref, o_ref):
  i = pl.program_id(0)

  @pl.when(i < HB // TM)
  def _():
    o_ref[...] = y1_ref[...]

  @pl.when(i >= HB // TM)
  def _():
    acc = lax.dot_general(
        x2_ref[...], w_ref[...],
        (((1,), (1,)), ((), ())),
        preferred_element_type=jnp.float32,
    )
    o_ref[...] = acc + b_ref[...]


def _linear2_tc(y1, x2, w, b2d):
  nh = HB // TM
  return pl.pallas_call(
      _mm2_body,
      grid=(B // TM,),
      in_specs=[
          pl.BlockSpec((TM, D), lambda i: (jnp.minimum(i, nh - 1), 0)),
          pl.BlockSpec((TM, D), lambda i: (jnp.maximum(i - nh, 0), 0)),
          pl.BlockSpec((D, D), lambda i: (0, 0)),
          pl.BlockSpec((1, D), lambda i: (0, 0)),
      ],
      out_specs=pl.BlockSpec((TM, D), lambda i: (i, 0)),
      out_shape=jax.ShapeDtypeStruct((B, D), jnp.float32),
  )(y1, x2, w, b2d)


@jax.jit
def kernel(h_dag, obs_ptr, W, b):
  ptr_pad = jnp.concatenate(
      [obs_ptr,
       jnp.full((PTR_PAD - (B + 1),), jnp.iinfo(jnp.int32).max, jnp.int32)])
  b2d = b.reshape(1, D)
  g1 = _segmax_sc(h_dag, ptr_pad, 0)
  g2 = _segmax_sc(h_dag, ptr_pad, HB)
  y1 = _linear_tc(g1, W, b2d)
  return _linear2_tc(y1, g2, W, b2d)


# speedup vs baseline: 38.8258x; 1.0040x over previous
import functools

import jax
import jax.numpy as jnp
from jax import lax
from jax.experimental import pallas as pl
from jax.experimental.pallas import tpu as pltpu
from jax.experimental.pallas import tpu_sc as plsc

N = 131072
B = 16384
D = 1024

NC = 2
NS = 16
NW = NC * NS
L = 16
CG = D // L
SBW = B // NW
CR = 48
OS = 16
PTR_CHUNK = 592
PTR_PAD = 16464
NPASS = 1


def _sread(ref, i):
  return ref[pl.ds(i, L)][0]


def _segmax_body(h_hbm, ptr_hbm, out_hbm, ptr_v, buf, oseg, sem):
  cid = lax.axis_index("c")
  sid = lax.axis_index("s")
  wid = sid * NC + cid
  s0 = wid * SBW

  pltpu.sync_copy(ptr_hbm.at[pl.ds(s0, PTR_CHUNK)], ptr_v)

  r0 = _sread(ptr_v, 0)
  r1 = _sread(ptr_v, SBW)

  neg = jnp.full((L,), -jnp.inf, dtype=jnp.float32)

  a0 = pl.multiple_of((r0 // 8) * 8, 8)
  nchunks = (r1 - a0 + CR - 1) // CR

  def chunk_base(c):
    return pl.multiple_of(jnp.minimum(a0 + c * CR, N - CR), 8)

  def chunk_dst(c):
    return buf.at[pl.ds(jnp.bitwise_and(c, 1) * CR, CR)]

  def issue(c):
    pltpu.async_copy(h_hbm.at[pl.ds(chunk_base(c), CR)], chunk_dst(c), sem)

  issue(jnp.int32(0))

  def chunk_body(c, s):
    pltpu.make_async_copy(
        h_hbm.at[pl.ds(chunk_base(c), CR)], chunk_dst(c), sem).wait()

    @pl.when(c + 1 < nchunks)
    def _():
      issue(c + 1)

    gridpos = a0 + c * CR
    rstart = jnp.maximum(r0, gridpos)
    rend = jnp.minimum(r1, gridpos + CR)
    boff = jnp.bitwise_and(c, 1) * CR + (rstart - chunk_base(c))

    lo = s
    vlo = _sread(ptr_v, s)
    for st in (32, 16, 8, 4, 2, 1):
      probe = lo + st
      pv = _sread(ptr_v, probe)
      ok = pv <= rend
      lo = jnp.where(ok, probe, lo)
      vlo = jnp.where(ok, pv, vlo)
    nb = lo - s
    nruns = nb + jnp.where(vlo < rend, jnp.int32(1), jnp.int32(0))

    def do_run(sk, q0, q1, pend, cont):
      t = jnp.bitwise_and(sk, OS - 1)
      row0 = boff + (q0 - rstart)
      nrows = q1 - q0

      for h in range(NPASS):
        hof = h * (CG // NPASS) * L
        npc = CG // NPASS
        if cont is None:
          acc = (neg,) * npc
        else:
          acc = tuple(
              jnp.where(cont, oseg[t, pl.ds(hof + cg * L, L)], neg)
              for cg in range(npc))

        def rb(i, a):
          return tuple(
              jnp.maximum(a[cg], buf[row0 + i, pl.ds(hof + cg * L, L)])
              for cg in range(npc))

        acc = lax.fori_loop(0, nrows, rb, acc)
        for cg in range(npc):
          oseg[t, pl.ds(hof + cg * L, L)] = acc[cg]

      done_group = jnp.logical_and(
          q1 == pend, jnp.bitwise_and(sk + 1, OS - 1) == 0)

      @pl.when(done_group)
      def _():
        gstart = pl.multiple_of(s0 + sk + 1 - OS, OS)
        pltpu.sync_copy(oseg, out_hbm.at[pl.ds(gstart, OS)])

    pstart0 = _sread(ptr_v, s)
    pend0 = _sread(ptr_v, s + 1)
    do_run(s, rstart, jnp.minimum(rend, pend0), pend0, pstart0 < rstart)

    def run_body(k, _):
      sk = s + k
      q0 = _sread(ptr_v, sk)
      pend = _sread(ptr_v, sk + 1)
      do_run(sk, q0, jnp.minimum(rend, pend), pend, None)
      return 0

    lax.fori_loop(1, nruns, run_body, 0)
    return s + nb

  lax.fori_loop(0, nchunks, chunk_body, jnp.int32(0))


def _segmax_sc(h_dag, ptr_pad):
  mesh = plsc.VectorSubcoreMesh(core_axis_name="c", subcore_axis_name="s")
  f = pl.kernel(
      _segmax_body,
      out_type=jax.ShapeDtypeStruct((B, D), jnp.float32),
      mesh=mesh,
      scratch_types=[
          pltpu.VMEM((PTR_CHUNK,), jnp.int32),
          pltpu.VMEM((2 * CR, D), jnp.float32),
          pltpu.VMEM((OS, D), jnp.float32),
          pltpu.SemaphoreType.DMA,
      ],
      name="segmax_sc",
  )
  return f(h_dag, ptr_pad)


TM = 1024


def _mm_body(x_ref, w_ref, b_ref, o_ref):
  acc = lax.dot_general(
      x_ref[...], w_ref[...],
      (((1,), (1,)), ((), ())),
      preferred_element_type=jnp.float32,
  )
  o_ref[...] = acc + b_ref[...]


def _linear_tc(x, w, b2d):
  return pl.pallas_call(
      _mm_body,
      grid=(B // TM,),
      in_specs=[
          pl.BlockSpec((TM, D), lambda i: (i, 0)),
          pl.BlockSpec((D, D), lambda i: (0, 0)),
          pl.BlockSpec((1, D), lambda i: (0, 0)),
      ],
      out_specs=pl.BlockSpec((TM, D), lambda i: (i, 0)),
      out_shape=jax.ShapeDtypeStruct((B, D), jnp.float32),
  )(x, w, b2d)


@jax.jit
def kernel(h_dag, obs_ptr, W, b):
  ptr_pad = jnp.concatenate(
      [obs_ptr,
       jnp.full((PTR_PAD - (B + 1),), jnp.iinfo(jnp.int32).max, jnp.int32)])
  h_glob = _segmax_sc(h_dag, ptr_pad)
  return _linear_tc(h_glob, W, b.reshape(1, D))


# speedup vs baseline: 42.3621x; 1.0911x over previous
import functools

import jax
import jax.numpy as jnp
from jax import lax
from jax.experimental import pallas as pl
from jax.experimental.pallas import tpu as pltpu
from jax.experimental.pallas import tpu_sc as plsc

N = 131072
B = 16384
D = 1024

NC = 2
NS = 16
NW = NC * NS
L = 16
CG = D // L
SBW = B // NW
CR = 24
NBUF = 4
OS = 16
PTR_CHUNK = 592
PTR_PAD = 16464
NPASS = 2


def _sread(ref, i):
  return ref[pl.ds(i, L)][0]


def _segmax_body(h_hbm, ptr_hbm, out_hbm, ptr_v, buf, oseg, sem):
  cid = lax.axis_index("c")
  sid = lax.axis_index("s")
  wid = sid * NC + cid
  s0 = wid * SBW

  pltpu.sync_copy(ptr_hbm.at[pl.ds(s0, PTR_CHUNK)], ptr_v)

  r0 = _sread(ptr_v, 0)
  r1 = _sread(ptr_v, SBW)

  neg = jnp.full((L,), -jnp.inf, dtype=jnp.float32)

  a0 = pl.multiple_of((r0 // 8) * 8, 8)
  nchunks = (r1 - a0 + CR - 1) // CR

  def chunk_base(c):
    return pl.multiple_of(jnp.minimum(a0 + c * CR, N - CR), 8)

  def chunk_dst(c):
    return buf.at[pl.ds(jnp.bitwise_and(c, NBUF - 1) * CR, CR)]

  def issue(c):
    pltpu.async_copy(h_hbm.at[pl.ds(chunk_base(c), CR)], chunk_dst(c),
                     sem.at[jnp.bitwise_and(c, NBUF - 1)])

  for j in range(NBUF - 1):
    @pl.when(j < nchunks)
    def _():
      issue(jnp.int32(j))

  def chunk_body(c, s):
    pltpu.make_async_copy(
        h_hbm.at[pl.ds(chunk_base(c), CR)], chunk_dst(c),
        sem.at[jnp.bitwise_and(c, NBUF - 1)]).wait()

    @pl.when(c + NBUF - 1 < nchunks)
    def _():
      issue(c + NBUF - 1)

    gridpos = a0 + c * CR
    rstart = jnp.maximum(r0, gridpos)
    rend = jnp.minimum(r1, gridpos + CR)
    boff = jnp.bitwise_and(c, NBUF - 1) * CR + (rstart - chunk_base(c))

    lo = s
    vlo = _sread(ptr_v, s)
    for st in (32, 16, 8, 4, 2, 1):
      probe = lo + st
      pv = _sread(ptr_v, probe)
      ok = pv <= rend
      lo = jnp.where(ok, probe, lo)
      vlo = jnp.where(ok, pv, vlo)
    nb = lo - s
    nruns = nb + jnp.where(vlo < rend, jnp.int32(1), jnp.int32(0))

    def do_run(sk, q0, q1, pend, cont):
      t = jnp.bitwise_and(sk, OS - 1)
      row0 = boff + (q0 - rstart)
      nrows = q1 - q0

      for h in range(NPASS):
        hof = h * (CG // NPASS) * L
        npc = CG // NPASS
        if cont is None:
          acc = (neg,) * npc
        else:
          acc = tuple(
              jnp.where(cont, oseg[t, pl.ds(hof + cg * L, L)], neg)
              for cg in range(npc))

        def rb(i, a):
          return tuple(
              jnp.maximum(a[cg], buf[row0 + i, pl.ds(hof + cg * L, L)])
              for cg in range(npc))

        acc = lax.fori_loop(0, nrows, rb, acc)
        for cg in range(npc):
          oseg[t, pl.ds(hof + cg * L, L)] = acc[cg]

      done_group = jnp.logical_and(
          q1 == pend, jnp.bitwise_and(sk + 1, OS - 1) == 0)

      @pl.when(done_group)
      def _():
        gstart = pl.multiple_of(s0 + sk + 1 - OS, OS)
        pltpu.sync_copy(oseg, out_hbm.at[pl.ds(gstart, OS)])

    pstart0 = _sread(ptr_v, s)
    pend0 = _sread(ptr_v, s + 1)
    do_run(s, rstart, jnp.minimum(rend, pend0), pend0, pstart0 < rstart)

    def run_body(k, _):
      sk = s + k
      q0 = _sread(ptr_v, sk)
      pend = _sread(ptr_v, sk + 1)
      do_run(sk, q0, jnp.minimum(rend, pend), pend, None)
      return 0

    lax.fori_loop(1, nruns, run_body, 0)
    return s + nb

  lax.fori_loop(0, nchunks, chunk_body, jnp.int32(0))


def _segmax_sc(h_dag, ptr_pad):
  mesh = plsc.VectorSubcoreMesh(core_axis_name="c", subcore_axis_name="s")
  f = pl.kernel(
      _segmax_body,
      out_type=jax.ShapeDtypeStruct((B, D), jnp.float32),
      mesh=mesh,
      scratch_types=[
          pltpu.VMEM((PTR_CHUNK,), jnp.int32),
          pltpu.VMEM((NBUF * CR, D), jnp.float32),
          pltpu.VMEM((OS, D), jnp.float32),
          pltpu.SemaphoreType.DMA((NBUF,)),
      ],
      name="segmax_sc",
  )
  return f(h_dag, ptr_pad)


TM = 1024


def _mm_body(x_ref, w_ref, b_ref, o_ref):
  acc = lax.dot_general(
      x_ref[...], w_ref[...],
      (((1,), (1,)), ((), ())),
      preferred_element_type=jnp.float32,
  )
  o_ref[...] = acc + b_ref[...]


def _linear_tc(x, w, b2d):
  return pl.pallas_call(
      _mm_body,
      grid=(B // TM,),
      in_specs=[
          pl.BlockSpec((TM, D), lambda i: (i, 0)),
          pl.BlockSpec((D, D), lambda i: (0, 0)),
          pl.BlockSpec((1, D), lambda i: (0, 0)),
      ],
      out_specs=pl.BlockSpec((TM, D), lambda i: (i, 0)),
      out_shape=jax.ShapeDtypeStruct((B, D), jnp.float32),
  )(x, w, b2d)


@jax.jit
def kernel(h_dag, obs_ptr, W, b):
  ptr_pad = jnp.concatenate(
      [obs_ptr,
       jnp.full((PTR_PAD - (B + 1),), jnp.iinfo(jnp.int32).max, jnp.int32)])
  h_glob = _segmax_sc(h_dag, ptr_pad)
  return _linear_tc(h_glob, W, b.reshape(1, D))


# speedup vs baseline: 43.3030x; 1.0222x over previous
import functools

import jax
import jax.numpy as jnp
from jax import lax
from jax.experimental import pallas as pl
from jax.experimental.pallas import tpu as pltpu
from jax.experimental.pallas import tpu_sc as plsc

N = 131072
B = 16384
D = 1024

NC = 2
NS = 16
NW = NC * NS
L = 16
CG = D // L
SBW = B // NW
CR = 32
NBUF = 3
OS = 16
PTR_CHUNK = 592
PTR_PAD = 16464
NPASS = 2


def _sread(ref, i):
  return ref[pl.ds(i, L)][0]


def _segmax_body(h_hbm, ptr_hbm, out_hbm, ptr_v, buf, oseg, sem):
  cid = lax.axis_index("c")
  sid = lax.axis_index("s")
  wid = sid * NC + cid
  s0 = wid * SBW

  pltpu.sync_copy(ptr_hbm.at[pl.ds(s0, PTR_CHUNK)], ptr_v)

  r0 = _sread(ptr_v, 0)
  r1 = _sread(ptr_v, SBW)

  neg = jnp.full((L,), -jnp.inf, dtype=jnp.float32)

  a0 = pl.multiple_of((r0 // 8) * 8, 8)
  nchunks = (r1 - a0 + CR - 1) // CR

  def chunk_base(c):
    return pl.multiple_of(jnp.minimum(a0 + c * CR, N - CR), 8)

  def chunk_dst(c):
    return buf.at[pl.ds(lax.rem(c, NBUF) * CR, CR)]

  def issue(c):
    pltpu.async_copy(h_hbm.at[pl.ds(chunk_base(c), CR)], chunk_dst(c),
                     sem.at[lax.rem(c, NBUF)])

  for j in range(NBUF - 1):
    @pl.when(j < nchunks)
    def _():
      issue(jnp.int32(j))

  def chunk_body(c, s):
    pltpu.make_async_copy(
        h_hbm.at[pl.ds(chunk_base(c), CR)], chunk_dst(c),
        sem.at[lax.rem(c, NBUF)]).wait()

    @pl.when(c + NBUF - 1 < nchunks)
    def _():
      issue(c + NBUF - 1)

    gridpos = a0 + c * CR
    rstart = jnp.maximum(r0, gridpos)
    rend = jnp.minimum(r1, gridpos + CR)
    boff = lax.rem(c, NBUF) * CR + (rstart - chunk_base(c))

    lo = s
    vlo = _sread(ptr_v, s)
    for st in (32, 16, 8, 4, 2, 1):
      probe = lo + st
      pv = _sread(ptr_v, probe)
      ok = pv <= rend
      lo = jnp.where(ok, probe, lo)
      vlo = jnp.where(ok, pv, vlo)
    nb = lo - s
    nruns = nb + jnp.where(vlo < rend, jnp.int32(1), jnp.int32(0))

    def do_run(sk, q0, q1, pend, cont):
      t = jnp.bitwise_and(sk, OS - 1)
      row0 = boff + (q0 - rstart)
      nrows = q1 - q0

      for h in range(NPASS):
        hof = h * (CG // NPASS) * L
        npc = CG // NPASS
        if cont is None:
          acc = (neg,) * npc
        else:
          acc = tuple(
              jnp.where(cont, oseg[t, pl.ds(hof + cg * L, L)], neg)
              for cg in range(npc))

        def rb(i, a):
          return tuple(
              jnp.maximum(a[cg], buf[row0 + i, pl.ds(hof + cg * L, L)])
              for cg in range(npc))

        acc = lax.fori_loop(0, nrows, rb, acc)
        for cg in range(npc):
          oseg[t, pl.ds(hof + cg * L, L)] = acc[cg]

      done_group = jnp.logical_and(
          q1 == pend, jnp.bitwise_and(sk + 1, OS - 1) == 0)

      @pl.when(done_group)
      def _():
        gstart = pl.multiple_of(s0 + sk + 1 - OS, OS)
        pltpu.sync_copy(oseg, out_hbm.at[pl.ds(gstart, OS)])

    pstart0 = _sread(ptr_v, s)
    pend0 = _sread(ptr_v, s + 1)
    do_run(s, rstart, jnp.minimum(rend, pend0), pend0, pstart0 < rstart)

    def run_body(k, _):
      sk = s + k
      q0 = _sread(ptr_v, sk)
      pend = _sread(ptr_v, sk + 1)
      do_run(sk, q0, jnp.minimum(rend, pend), pend, None)
      return 0

    lax.fori_loop(1, nruns, run_body, 0)
    return s + nb

  lax.fori_loop(0, nchunks, chunk_body, jnp.int32(0))


def _segmax_sc(h_dag, ptr_pad):
  mesh = plsc.VectorSubcoreMesh(core_axis_name="c", subcore_axis_name="s")
  f = pl.kernel(
      _segmax_body,
      out_type=jax.ShapeDtypeStruct((B, D), jnp.float32),
      mesh=mesh,
      scratch_types=[
          pltpu.VMEM((PTR_CHUNK,), jnp.int32),
          pltpu.VMEM((NBUF * CR, D), jnp.float32),
          pltpu.VMEM((OS, D), jnp.float32),
          pltpu.SemaphoreType.DMA((NBUF,)),
      ],
      name="segmax_sc",
  )
  return f(h_dag, ptr_pad)


TM = 1024


def _mm_body(x_ref, w_ref, b_ref, o_ref):
  acc = lax.dot_general(
      x_ref[...].astype(jnp.bfloat16), w_ref[...].astype(jnp.bfloat16),
      (((1,), (1,)), ((), ())),
      preferred_element_type=jnp.float32,
  )
  o_ref[...] = acc + b_ref[...]


def _linear_tc(x, w, b2d):
  return pl.pallas_call(
      _mm_body,
      grid=(B // TM,),
      in_specs=[
          pl.BlockSpec((TM, D), lambda i: (i, 0)),
          pl.BlockSpec((D, D), lambda i: (0, 0)),
          pl.BlockSpec((1, D), lambda i: (0, 0)),
      ],
      out_specs=pl.BlockSpec((TM, D), lambda i: (i, 0)),
      out_shape=jax.ShapeDtypeStruct((B, D), jnp.float32),
  )(x, w, b2d)


@jax.jit
def kernel(h_dag, obs_ptr, W, b):
  ptr_pad = jnp.concatenate(
      [obs_ptr,
       jnp.full((PTR_PAD - (B + 1),), jnp.iinfo(jnp.int32).max, jnp.int32)])
  h_glob = _segmax_sc(h_dag, ptr_pad)
  return _linear_tc(h_glob, W, b.reshape(1, D))


# speedup vs baseline: 46.1543x; 1.0658x over previous
import functools

import jax
import jax.numpy as jnp
from jax import lax
from jax.experimental import pallas as pl
from jax.experimental.pallas import tpu as pltpu
from jax.experimental.pallas import tpu_sc as plsc

N = 131072
B = 16384
D = 1024

NC = 2
NS = 16
NW = NC * NS
L = 16
CG = D // L
SBW = B // NW
CR = 32
NBUF = 3
OS = 8
PTR_CHUNK = 592
PTR_PAD = 16464
NPASS = 2


def _sread(ref, i):
  return ref[pl.ds(i, L)][0]


def _segmax_body(h_hbm, ptr_hbm, out_hbm, ptr_v, buf, oseg, sem, osem):
  cid = lax.axis_index("c")
  sid = lax.axis_index("s")
  wid = sid * NC + cid
  s0 = wid * SBW

  pltpu.sync_copy(ptr_hbm.at[pl.ds(s0, PTR_CHUNK)], ptr_v)

  r0 = _sread(ptr_v, 0)
  r1 = _sread(ptr_v, SBW)

  neg = jnp.full((L,), -jnp.inf, dtype=jnp.float32)

  a0 = pl.multiple_of((r0 // 8) * 8, 8)
  nchunks = (r1 - a0 + CR - 1) // CR

  def chunk_base(c):
    return pl.multiple_of(jnp.minimum(a0 + c * CR, N - CR), 8)

  def chunk_dst(c):
    return buf.at[pl.ds(lax.rem(c, NBUF) * CR, CR)]

  def issue(c):
    pltpu.async_copy(h_hbm.at[pl.ds(chunk_base(c), CR)], chunk_dst(c),
                     sem.at[lax.rem(c, NBUF)])

  for j in range(NBUF - 1):
    @pl.when(j < nchunks)
    def _():
      issue(jnp.int32(j))

  def chunk_body(c, s):
    pltpu.make_async_copy(
        h_hbm.at[pl.ds(chunk_base(c), CR)], chunk_dst(c),
        sem.at[lax.rem(c, NBUF)]).wait()

    @pl.when(c + NBUF - 1 < nchunks)
    def _():
      issue(c + NBUF - 1)

    gridpos = a0 + c * CR
    rstart = jnp.maximum(r0, gridpos)
    rend = jnp.minimum(r1, gridpos + CR)
    boff = lax.rem(c, NBUF) * CR + (rstart - chunk_base(c))

    lo = s
    vlo = _sread(ptr_v, s)
    for st in (32, 16, 8, 4, 2, 1):
      probe = lo + st
      pv = _sread(ptr_v, probe)
      ok = pv <= rend
      lo = jnp.where(ok, probe, lo)
      vlo = jnp.where(ok, pv, vlo)
    nb = lo - s
    nruns = nb + jnp.where(vlo < rend, jnp.int32(1), jnp.int32(0))

    def do_run(sk, q0, q1, pend, cont):
      t = jnp.bitwise_and(sk, 2 * OS - 1)
      row0 = boff + (q0 - rstart)
      nrows = q1 - q0

      for h in range(NPASS):
        hof = h * (CG // NPASS) * L
        npc = CG // NPASS
        if cont is None:
          acc = (neg,) * npc
        else:
          acc = tuple(
              jnp.where(cont, oseg[t, pl.ds(hof + cg * L, L)], neg)
              for cg in range(npc))

        def rb(i, a):
          return tuple(
              jnp.maximum(a[cg], buf[row0 + i, pl.ds(hof + cg * L, L)])
              for cg in range(npc))

        acc = lax.fori_loop(0, nrows, rb, acc)
        for cg in range(npc):
          oseg[t, pl.ds(hof + cg * L, L)] = acc[cg]

      done_group = jnp.logical_and(
          q1 == pend, jnp.bitwise_and(sk + 1, OS - 1) == 0)

      @pl.when(done_group)
      def _():
        fb = pl.multiple_of(jnp.bitwise_and(sk + 1 - OS, 2 * OS - 1), OS)
        gstart = pl.multiple_of(s0 + sk + 1 - OS, OS)

        @pl.when(sk + 1 >= 2 * OS)
        def _():
          pltpu.make_async_copy(
              oseg.at[pl.ds(
                  pl.multiple_of(jnp.bitwise_and(fb + OS, 2 * OS - 1), OS),
                  OS)],
              out_hbm.at[pl.ds(pl.multiple_of(s0, OS), OS)], osem).wait()

        pltpu.async_copy(
            oseg.at[pl.ds(fb, OS)], out_hbm.at[pl.ds(gstart, OS)], osem)

    pstart0 = _sread(ptr_v, s)
    pend0 = _sread(ptr_v, s + 1)
    do_run(s, rstart, jnp.minimum(rend, pend0), pend0, pstart0 < rstart)

    def run_body(k, _):
      sk = s + k
      q0 = _sread(ptr_v, sk)
      pend = _sread(ptr_v, sk + 1)
      do_run(sk, q0, jnp.minimum(rend, pend), pend, None)
      return 0

    lax.fori_loop(1, nruns, run_body, 0)
    return s + nb

  lax.fori_loop(0, nchunks, chunk_body, jnp.int32(0))

  pltpu.make_async_copy(
      oseg.at[pl.ds(0, OS)],
      out_hbm.at[pl.ds(pl.multiple_of(s0, OS), OS)], osem).wait()


def _segmax_sc(h_dag, ptr_pad):
  mesh = plsc.VectorSubcoreMesh(core_axis_name="c", subcore_axis_name="s")
  f = pl.kernel(
      _segmax_body,
      out_type=jax.ShapeDtypeStruct((B, D), jnp.float32),
      mesh=mesh,
      scratch_types=[
          pltpu.VMEM((PTR_CHUNK,), jnp.int32),
          pltpu.VMEM((NBUF * CR, D), jnp.float32),
          pltpu.VMEM((2 * OS, D), jnp.float32),
          pltpu.SemaphoreType.DMA((NBUF,)),
          pltpu.SemaphoreType.DMA,
      ],
      name="segmax_sc",
  )
  return f(h_dag, ptr_pad)


TM = 1024


def _mm_body(x_ref, w_ref, b_ref, o_ref):
  acc = lax.dot_general(
      x_ref[...].astype(jnp.bfloat16), w_ref[...].astype(jnp.bfloat16),
      (((1,), (1,)), ((), ())),
      preferred_element_type=jnp.float32,
  )
  o_ref[...] = acc + b_ref[...]


def _linear_tc(x, w, b2d):
  return pl.pallas_call(
      _mm_body,
      grid=(B // TM,),
      in_specs=[
          pl.BlockSpec((TM, D), lambda i: (i, 0)),
          pl.BlockSpec((D, D), lambda i: (0, 0)),
          pl.BlockSpec((1, D), lambda i: (0, 0)),
      ],
      out_specs=pl.BlockSpec((TM, D), lambda i: (i, 0)),
      out_shape=jax.ShapeDtypeStruct((B, D), jnp.float32),
  )(x, w, b2d)


@jax.jit
def kernel(h_dag, obs_ptr, W, b):
  ptr_pad = jnp.concatenate(
      [obs_ptr,
       jnp.full((PTR_PAD - (B + 1),), jnp.iinfo(jnp.int32).max, jnp.int32)])
  h_glob = _segmax_sc(h_dag, ptr_pad)
  return _linear_tc(h_glob, W, b.reshape(1, D))
